# baseline (speedup 1.0000x reference)
import jax
import jax.numpy as jnp
from jax.experimental import pallas as pl
from jax.experimental.pallas import tpu as pltpu

_EPS_NORM = 1e-12
_EPS_D2 = 1e-12
_EPS_LOG = 1e-08
_LOG2E = 1.4426950408889634


def _proj_body(a_ref, b_ref, wa_ref, ba_ref, wb_ref, bb_ref, c_ref,
               za_ref, zb_ref, y2_ref):
    xa = jax.lax.dot_general(a_ref[...], wa_ref[...], (((1,), (1,)), ((), ())),
                             preferred_element_type=jnp.float32) + ba_ref[...]
    na = jnp.sqrt(jnp.sum(xa * xa, axis=-1, keepdims=True))
    za_ref[...] = xa / jnp.maximum(na, _EPS_NORM)
    xb = jax.lax.dot_general(b_ref[...], wb_ref[...], (((1,), (1,)), ((), ())),
                             preferred_element_type=jnp.float32) + bb_ref[...]
    nb = jnp.sqrt(jnp.sum(xb * xb, axis=-1, keepdims=True))
    zb_ref[...] = xb / jnp.maximum(nb, _EPS_NORM)

    @pl.when(pl.program_id(0) == 0)
    def _y2():
        c = c_ref[...]
        ones_m = jnp.ones((8, c.shape[1]), jnp.float32)
        y2_ref[...] = jax.lax.dot_general(
            ones_m, c * c, (((1,), (1,)), ((), ())),
            preferred_element_type=jnp.float32)[:1] + 1.0


def _main_body(za_ref, zb_ref, zbf_ref, c_ref, y2_ref, out_ref,
               avg_a_acc, avg_b_acc, colsum_acc,
               d2_acc, match_acc, lse_acc, diag_acc):
    i = pl.program_id(0)
    nsteps = pl.num_programs(0)
    bsz = za_ref.shape[0] * nsteps

    @pl.when(i == 0)
    def _init():
        avg_a_acc[...] = jnp.zeros_like(avg_a_acc)
        avg_b_acc[...] = jnp.zeros_like(avg_b_acc)
        colsum_acc[...] = jnp.zeros_like(colsum_acc)
        d2_acc[...] = jnp.zeros_like(d2_acc)
        match_acc[...] = jnp.zeros_like(match_acc)
        lse_acc[...] = jnp.zeros_like(lse_acc)
        diag_acc[...] = jnp.zeros_like(diag_acc)

    y2 = y2_ref[...]
    za = za_ref[...]
    zb = zb_ref[...]
    ones_m8 = jnp.ones((8, za.shape[0]), jnp.float32)

    def side(z, avg_acc):
        g2 = jax.lax.dot_general(z * (-2.0), c_ref[...],
                                 (((1,), (1,)), ((), ())),
                                 preferred_element_type=jnp.float32)
        d2c = jnp.maximum(y2 + g2, _EPS_D2)
        m = jnp.min(d2c, axis=-1, keepdims=True)
        e = jnp.exp2(d2c * jax.lax.rsqrt(d2c) * (-5.0 * _LOG2E))
        s = jnp.sum(e, axis=-1, keepdims=True)
        rt = jnp.transpose(1.0 / s, (1, 0))
        avg_acc[...] += jax.lax.dot_general(
            rt, e, (((1,), (0,)), ((), ())),
            preferred_element_type=jnp.float32)
        return d2c, m

    d2c_a, m_a = side(za, avg_a_acc)
    d2c_b, m_b = side(zb, avg_b_acc)

    d2_acc[...] += jnp.sum(m_a + m_b, keepdims=True)
    ov = jnp.min(jnp.maximum(d2c_a - m_a, d2c_b - m_b), axis=-1,
                 keepdims=True)
    match_acc[...] += jnp.sum((ov <= 0.0).astype(jnp.float32), keepdims=True)

    es = jnp.exp2(jax.lax.dot_general(
        za * jnp.float32(_LOG2E / 0.07), zbf_ref[...],
        (((1,), (1,)), ((), ())), preferred_element_type=jnp.float32))
    rowsum = jnp.sum(es, axis=-1, keepdims=True)
    lse_acc[...] += jnp.sum(jnp.log(rowsum), keepdims=True)
    colsum_acc[...] += jax.lax.dot_general(
        ones_m8, es, (((1,), (0,)), ((), ())),
        preferred_element_type=jnp.float32)[:1]
    diag_acc[...] += jnp.sum(za * zb, keepdims=True) / 0.07

    @pl.when(i == nsteps - 1)
    def _fin():
        avg_a = avg_a_acc[...] * (1.0 / bsz)
        avg_b = avg_b_acc[...] * (1.0 / bsz)
        ent_a = -jnp.sum(avg_a * jnp.log(avg_a + _EPS_LOG), keepdims=True)
        ent_b = -jnp.sum(avg_b * jnp.log(avg_b + _EPS_LOG), keepdims=True)
        col_lse = jnp.sum(jnp.log(colsum_acc[...]), keepdims=True)
        lane = jax.lax.broadcasted_iota(jnp.int32, out_ref.shape, 1)
        out_ref[...] = (jnp.where(lane == 0, d2_acc[...], 0.0)
                        + jnp.where(lane == 1, match_acc[...], 0.0)
                        + jnp.where(lane == 2, ent_a, 0.0)
                        + jnp.where(lane == 3, ent_b, 0.0)
                        + jnp.where(lane == 4, lse_acc[...], 0.0)
                        + jnp.where(lane == 5, col_lse, 0.0)
                        + jnp.where(lane == 6, diag_acc[...], 0.0))


def kernel(a, b, Wa, ba, Wb, bb, codebook):
    bsz, dim_a = a.shape
    dim_b = b.shape[1]
    emb = Wa.shape[0]
    nc = codebook.shape[0]
    f32 = jnp.float32

    tr1 = 512
    za, zb, y2p1 = pl.pallas_call(
        _proj_body,
        grid=(bsz // tr1,),
        in_specs=[
            pl.BlockSpec((tr1, dim_a), lambda i: (i, 0)),
            pl.BlockSpec((tr1, dim_b), lambda i: (i, 0)),
            pl.BlockSpec((emb, dim_a), lambda i: (0, 0)),
            pl.BlockSpec((1, emb), lambda i: (0, 0)),
            pl.BlockSpec((emb, dim_b), lambda i: (0, 0)),
            pl.BlockSpec((1, emb), lambda i: (0, 0)),
            pl.BlockSpec((nc, emb), lambda i: (0, 0)),
        ],
        out_specs=[
            pl.BlockSpec((tr1, emb), lambda i: (i, 0)),
            pl.BlockSpec((tr1, emb), lambda i: (i, 0)),
            pl.BlockSpec((1, nc), lambda i: (0, 0)),
        ],
        out_shape=[
            jax.ShapeDtypeStruct((bsz, emb), f32),
            jax.ShapeDtypeStruct((bsz, emb), f32),
            jax.ShapeDtypeStruct((1, nc), f32),
        ],
    )(a, b, Wa, ba.reshape(1, emb), Wb, bb.reshape(1, emb), codebook)

    tr2 = 256
    main_out = pl.pallas_call(
        _main_body,
        grid=(bsz // tr2,),
        in_specs=[
            pl.BlockSpec((tr2, emb), lambda i: (i, 0)),
            pl.BlockSpec((tr2, emb), lambda i: (i, 0)),
            pl.BlockSpec((bsz, emb), lambda i: (0, 0)),
            pl.BlockSpec((nc, emb), lambda i: (0, 0)),
            pl.BlockSpec((1, nc), lambda i: (0, 0)),
        ],
        out_specs=pl.BlockSpec((1, 128), lambda i: (0, 0)),
        out_shape=jax.ShapeDtypeStruct((1, 128), f32),
        scratch_shapes=[
            pltpu.VMEM((1, nc), f32),
            pltpu.VMEM((1, nc), f32),
            pltpu.VMEM((1, bsz), f32),
            pltpu.VMEM((1, 1), f32),
            pltpu.VMEM((1, 1), f32),
            pltpu.VMEM((1, 1), f32),
            pltpu.VMEM((1, 1), f32),
        ],
    )(za, zb, zb, codebook, y2p1)

    d2min_sum = main_out[0, 0]
    match_sum = main_out[0, 1]
    ent_a = main_out[0, 2]
    ent_b = main_out[0, 3]
    lse_row_sum = main_out[0, 4]
    col_lse_sum = main_out[0, 5]
    diag_sum = main_out[0, 6]

    rec = 1.25 * d2min_sum / (bsz * emb)
    cm = 0.5 * (lse_row_sum + col_lse_sum) / bsz - diag_sum / bsz
    div = 0.5 * (ent_a + ent_b)
    loss = rec + 0.5 * cm - 0.1 * div
    match = match_sum / bsz
    return (loss, match)


# speedup vs baseline: 1.0065x; 1.0065x over previous
import jax
import jax.numpy as jnp
from jax.experimental import pallas as pl
from jax.experimental.pallas import tpu as pltpu

_EPS_NORM = 1e-12
_EPS_D2 = 1e-12
_EPS_LOG = 1e-08
_LOG2E = 1.4426950408889634


def _proj_body(a_ref, b_ref, wa_ref, ba_ref, wb_ref, bb_ref, za_ref, zb_ref):
    xa = jax.lax.dot_general(a_ref[...], wa_ref[...], (((1,), (1,)), ((), ())),
                             preferred_element_type=jnp.float32) + ba_ref[...]
    na = jnp.sqrt(jnp.sum(xa * xa, axis=-1, keepdims=True))
    za_ref[...] = xa / jnp.maximum(na, _EPS_NORM)
    xb = jax.lax.dot_general(b_ref[...], wb_ref[...], (((1,), (1,)), ((), ())),
                             preferred_element_type=jnp.float32) + bb_ref[...]
    nb = jnp.sqrt(jnp.sum(xb * xb, axis=-1, keepdims=True))
    zb_ref[...] = xb / jnp.maximum(nb, _EPS_NORM)


def _main_body(za_ref, zb_ref, zbf_ref, c_ref, out_ref,
               y2_acc, avg_a_acc, avg_b_acc, colsum_acc,
               d2_acc, match_acc, lse_acc, diag_acc):
    i = pl.program_id(0)
    nsteps = pl.num_programs(0)
    bsz = za_ref.shape[0] * nsteps

    @pl.when(i == 0)
    def _init():
        c = c_ref[...]
        ones_m = jnp.ones((8, c.shape[1]), jnp.float32)
        y2_acc[...] = jax.lax.dot_general(
            ones_m, c * c, (((1,), (1,)), ((), ())),
            preferred_element_type=jnp.float32)[:1] + 1.0
        avg_a_acc[...] = jnp.zeros_like(avg_a_acc)
        avg_b_acc[...] = jnp.zeros_like(avg_b_acc)
        colsum_acc[...] = jnp.zeros_like(colsum_acc)
        d2_acc[...] = jnp.zeros_like(d2_acc)
        match_acc[...] = jnp.zeros_like(match_acc)
        lse_acc[...] = jnp.zeros_like(lse_acc)
        diag_acc[...] = jnp.zeros_like(diag_acc)

    y2 = y2_acc[...]
    za = za_ref[...]
    zb = zb_ref[...]
    ones_m8 = jnp.ones((8, za.shape[0]), jnp.float32)

    def side(z, avg_acc):
        g2 = jax.lax.dot_general(z * (-2.0), c_ref[...],
                                 (((1,), (1,)), ((), ())),
                                 preferred_element_type=jnp.float32)
        d2c = jnp.maximum(y2 + g2, _EPS_D2)
        m = jnp.min(d2c, axis=-1, keepdims=True)
        e = jnp.exp2(d2c * jax.lax.rsqrt(d2c) * (-5.0 * _LOG2E))
        s = jnp.sum(e, axis=-1, keepdims=True)
        rt = jnp.transpose(1.0 / s, (1, 0))
        avg_acc[...] += jax.lax.dot_general(
            rt, e, (((1,), (0,)), ((), ())),
            preferred_element_type=jnp.float32)
        return d2c, m

    d2c_a, m_a = side(za, avg_a_acc)
    d2c_b, m_b = side(zb, avg_b_acc)

    d2_acc[...] += jnp.sum(m_a + m_b, keepdims=True)
    ov = jnp.min(jnp.maximum(d2c_a - m_a, d2c_b - m_b), axis=-1,
                 keepdims=True)
    match_acc[...] += jnp.sum((ov <= 0.0).astype(jnp.float32), keepdims=True)

    es = jnp.exp2(jax.lax.dot_general(
        za * jnp.float32(_LOG2E / 0.07), zbf_ref[...],
        (((1,), (1,)), ((), ())), preferred_element_type=jnp.float32))
    rowsum = jnp.sum(es, axis=-1, keepdims=True)
    lse_acc[...] += jnp.sum(jnp.log(rowsum), keepdims=True)
    colsum_acc[...] += jax.lax.dot_general(
        ones_m8, es, (((1,), (0,)), ((), ())),
        preferred_element_type=jnp.float32)[:1]
    diag_acc[...] += jnp.sum(za * zb, keepdims=True) / 0.07

    @pl.when(i == nsteps - 1)
    def _fin():
        avg_a = avg_a_acc[...] * (1.0 / bsz)
        avg_b = avg_b_acc[...] * (1.0 / bsz)
        ent_a = -jnp.sum(avg_a * jnp.log(avg_a + _EPS_LOG), keepdims=True)
        ent_b = -jnp.sum(avg_b * jnp.log(avg_b + _EPS_LOG), keepdims=True)
        col_lse = jnp.sum(jnp.log(colsum_acc[...]), keepdims=True)
        lane = jax.lax.broadcasted_iota(jnp.int32, out_ref.shape, 1)
        out_ref[...] = (jnp.where(lane == 0, d2_acc[...], 0.0)
                        + jnp.where(lane == 1, match_acc[...], 0.0)
                        + jnp.where(lane == 2, ent_a, 0.0)
                        + jnp.where(lane == 3, ent_b, 0.0)
                        + jnp.where(lane == 4, lse_acc[...], 0.0)
                        + jnp.where(lane == 5, col_lse, 0.0)
                        + jnp.where(lane == 6, diag_acc[...], 0.0))


def kernel(a, b, Wa, ba, Wb, bb, codebook):
    bsz, dim_a = a.shape
    dim_b = b.shape[1]
    emb = Wa.shape[0]
    nc = codebook.shape[0]
    f32 = jnp.float32

    tr1 = 512
    za, zb = pl.pallas_call(
        _proj_body,
        grid=(bsz // tr1,),
        in_specs=[
            pl.BlockSpec((tr1, dim_a), lambda i: (i, 0)),
            pl.BlockSpec((tr1, dim_b), lambda i: (i, 0)),
            pl.BlockSpec((emb, dim_a), lambda i: (0, 0)),
            pl.BlockSpec((1, emb), lambda i: (0, 0)),
            pl.BlockSpec((emb, dim_b), lambda i: (0, 0)),
            pl.BlockSpec((1, emb), lambda i: (0, 0)),
        ],
        out_specs=[
            pl.BlockSpec((tr1, emb), lambda i: (i, 0)),
            pl.BlockSpec((tr1, emb), lambda i: (i, 0)),
        ],
        out_shape=[
            jax.ShapeDtypeStruct((bsz, emb), f32),
            jax.ShapeDtypeStruct((bsz, emb), f32),
        ],
    )(a, b, Wa, ba.reshape(1, emb), Wb, bb.reshape(1, emb))

    tr2 = 256
    main_out = pl.pallas_call(
        _main_body,
        grid=(bsz // tr2,),
        in_specs=[
            pl.BlockSpec((tr2, emb), lambda i: (i, 0)),
            pl.BlockSpec((tr2, emb), lambda i: (i, 0)),
            pl.BlockSpec((bsz, emb), lambda i: (0, 0)),
            pl.BlockSpec((nc, emb), lambda i: (0, 0)),
        ],
        out_specs=pl.BlockSpec((1, 128), lambda i: (0, 0)),
        out_shape=jax.ShapeDtypeStruct((1, 128), f32),
        scratch_shapes=[
            pltpu.VMEM((1, nc), f32),
            pltpu.VMEM((1, nc), f32),
            pltpu.VMEM((1, nc), f32),
            pltpu.VMEM((1, bsz), f32),
            pltpu.VMEM((1, 1), f32),
            pltpu.VMEM((1, 1), f32),
            pltpu.VMEM((1, 1), f32),
            pltpu.VMEM((1, 1), f32),
        ],
    )(za, zb, zb, codebook)

    d2min_sum = main_out[0, 0]
    match_sum = main_out[0, 1]
    ent_a = main_out[0, 2]
    ent_b = main_out[0, 3]
    lse_row_sum = main_out[0, 4]
    col_lse_sum = main_out[0, 5]
    diag_sum = main_out[0, 6]

    rec = 1.25 * d2min_sum / (bsz * emb)
    cm = 0.5 * (lse_row_sum + col_lse_sum) / bsz - diag_sum / bsz
    div = 0.5 * (ent_a + ent_b)
    loss = rec + 0.5 * cm - 0.1 * div
    match = match_sum / bsz
    return (loss, match)


# speedup vs baseline: 1.0452x; 1.0384x over previous
import jax
import jax.numpy as jnp
from jax.experimental import pallas as pl
from jax.experimental.pallas import tpu as pltpu

_EPS_NORM = 1e-12
_EPS_D2 = 1e-12
_EPS_LOG = 1e-08
_LOG2E = 1.4426950408889634


def _proj_body(a_ref, b_ref, wa_ref, ba_ref, wb_ref, bb_ref, za_ref, zb_ref):
    xa = jax.lax.dot_general(a_ref[...], wa_ref[...], (((1,), (1,)), ((), ())),
                             preferred_element_type=jnp.float32) + ba_ref[...]
    na = jnp.sqrt(jnp.sum(xa * xa, axis=-1, keepdims=True))
    za_ref[...] = xa / jnp.maximum(na, _EPS_NORM)
    xb = jax.lax.dot_general(b_ref[...], wb_ref[...], (((1,), (1,)), ((), ())),
                             preferred_element_type=jnp.float32) + bb_ref[...]
    nb = jnp.sqrt(jnp.sum(xb * xb, axis=-1, keepdims=True))
    zb_ref[...] = xb / jnp.maximum(nb, _EPS_NORM)


def _main_body(za_ref, zb_ref, zbf_ref, c_ref, out_ref,
               y2_acc, avg_a_acc, avg_b_acc, colsum_acc,
               d2_acc, match_acc, lse_acc, diag_acc):
    i = pl.program_id(0)
    nsteps = pl.num_programs(0)
    bsz = za_ref.shape[0] * nsteps

    @pl.when(i == 0)
    def _init():
        c = c_ref[...]
        ones_m = jnp.ones((8, c.shape[1]), jnp.float32)
        y2_acc[...] = jax.lax.dot_general(
            ones_m, c * c, (((1,), (1,)), ((), ())),
            preferred_element_type=jnp.float32)[:1] + 1.0
        avg_a_acc[...] = jnp.zeros_like(avg_a_acc)
        avg_b_acc[...] = jnp.zeros_like(avg_b_acc)
        colsum_acc[...] = jnp.zeros_like(colsum_acc)
        d2_acc[...] = jnp.zeros_like(d2_acc)
        match_acc[...] = jnp.zeros_like(match_acc)
        lse_acc[...] = jnp.zeros_like(lse_acc)
        diag_acc[...] = jnp.zeros_like(diag_acc)

    y2 = y2_acc[...]
    za = za_ref[...]
    zb = zb_ref[...]
    ones_m8 = jnp.ones((8, za.shape[0]), jnp.float32)

    def side(z, avg_acc):
        g2 = jax.lax.dot_general(z * (-2.0), c_ref[...],
                                 (((1,), (1,)), ((), ())),
                                 preferred_element_type=jnp.float32)
        d2c = jnp.maximum(y2 + g2, _EPS_D2)
        m = jnp.min(d2c, axis=-1, keepdims=True)
        e = jnp.exp2(d2c * jax.lax.rsqrt(d2c) * (-5.0 * _LOG2E))
        s = jnp.sum(e, axis=-1, keepdims=True)
        rt = jnp.transpose(1.0 / s, (1, 0))
        avg_acc[...] += jax.lax.dot_general(
            rt, e, (((1,), (0,)), ((), ())),
            preferred_element_type=jnp.float32)
        return d2c, m

    d2c_a, m_a = side(za, avg_a_acc)
    d2c_b, m_b = side(zb, avg_b_acc)

    d2_acc[...] += jnp.sum(m_a + m_b, keepdims=True)
    ov = jnp.min(jnp.maximum(d2c_a - m_a, d2c_b - m_b), axis=-1,
                 keepdims=True)
    match_acc[...] += jnp.sum((ov <= 0.0).astype(jnp.float32), keepdims=True)

    es = jnp.exp2(jax.lax.dot_general(
        za * jnp.float32(_LOG2E / 0.07), zbf_ref[...],
        (((1,), (1,)), ((), ())), preferred_element_type=jnp.float32))
    rowsum = jnp.sum(es, axis=-1, keepdims=True)
    lse_acc[...] += jnp.sum(jnp.log(rowsum), keepdims=True)
    colsum_acc[...] += jax.lax.dot_general(
        ones_m8, es, (((1,), (0,)), ((), ())),
        preferred_element_type=jnp.float32)[:1]
    diag_acc[...] += jnp.sum(za * zb, keepdims=True) / 0.07

    @pl.when(i == nsteps - 1)
    def _fin():
        avg_a = avg_a_acc[...] * (1.0 / bsz)
        avg_b = avg_b_acc[...] * (1.0 / bsz)
        ent_a = -jnp.sum(avg_a * jnp.log(avg_a + _EPS_LOG), keepdims=True)
        ent_b = -jnp.sum(avg_b * jnp.log(avg_b + _EPS_LOG), keepdims=True)
        col_lse = jnp.sum(jnp.log(colsum_acc[...]), keepdims=True)
        lane = jax.lax.broadcasted_iota(jnp.int32, out_ref.shape, 1)
        out_ref[...] = (jnp.where(lane == 0, d2_acc[...], 0.0)
                        + jnp.where(lane == 1, match_acc[...], 0.0)
                        + jnp.where(lane == 2, ent_a, 0.0)
                        + jnp.where(lane == 3, ent_b, 0.0)
                        + jnp.where(lane == 4, lse_acc[...], 0.0)
                        + jnp.where(lane == 5, col_lse, 0.0)
                        + jnp.where(lane == 6, diag_acc[...], 0.0))


def kernel(a, b, Wa, ba, Wb, bb, codebook):
    bsz, dim_a = a.shape
    dim_b = b.shape[1]
    emb = Wa.shape[0]
    nc = codebook.shape[0]
    f32 = jnp.float32

    tr1 = 1024
    za, zb = pl.pallas_call(
        _proj_body,
        grid=(bsz // tr1,),
        in_specs=[
            pl.BlockSpec((tr1, dim_a), lambda i: (i, 0)),
            pl.BlockSpec((tr1, dim_b), lambda i: (i, 0)),
            pl.BlockSpec((emb, dim_a), lambda i: (0, 0)),
            pl.BlockSpec((1, emb), lambda i: (0, 0)),
            pl.BlockSpec((emb, dim_b), lambda i: (0, 0)),
            pl.BlockSpec((1, emb), lambda i: (0, 0)),
        ],
        out_specs=[
            pl.BlockSpec((tr1, emb), lambda i: (i, 0)),
            pl.BlockSpec((tr1, emb), lambda i: (i, 0)),
        ],
        out_shape=[
            jax.ShapeDtypeStruct((bsz, emb), f32),
            jax.ShapeDtypeStruct((bsz, emb), f32),
        ],
    )(a, b, Wa, ba.reshape(1, emb), Wb, bb.reshape(1, emb))

    tr2 = 256
    main_out = pl.pallas_call(
        _main_body,
        grid=(bsz // tr2,),
        in_specs=[
            pl.BlockSpec((tr2, emb), lambda i: (i, 0)),
            pl.BlockSpec((tr2, emb), lambda i: (i, 0)),
            pl.BlockSpec((bsz, emb), lambda i: (0, 0)),
            pl.BlockSpec((nc, emb), lambda i: (0, 0)),
        ],
        out_specs=pl.BlockSpec((1, 128), lambda i: (0, 0)),
        out_shape=jax.ShapeDtypeStruct((1, 128), f32),
        scratch_shapes=[
            pltpu.VMEM((1, nc), f32),
            pltpu.VMEM((1, nc), f32),
            pltpu.VMEM((1, nc), f32),
            pltpu.VMEM((1, bsz), f32),
            pltpu.VMEM((1, 1), f32),
            pltpu.VMEM((1, 1), f32),
            pltpu.VMEM((1, 1), f32),
            pltpu.VMEM((1, 1), f32),
        ],
    )(za, zb, zb, codebook)

    d2min_sum = main_out[0, 0]
    match_sum = main_out[0, 1]
    ent_a = main_out[0, 2]
    ent_b = main_out[0, 3]
    lse_row_sum = main_out[0, 4]
    col_lse_sum = main_out[0, 5]
    diag_sum = main_out[0, 6]

    rec = 1.25 * d2min_sum / (bsz * emb)
    cm = 0.5 * (lse_row_sum + col_lse_sum) / bsz - diag_sum / bsz
    div = 0.5 * (ent_a + ent_b)
    loss = rec + 0.5 * cm - 0.1 * div
    match = match_sum / bsz
    return (loss, match)


# speedup vs baseline: 1.0757x; 1.0292x over previous
import jax
import jax.numpy as jnp
from jax.experimental import pallas as pl
from jax.experimental.pallas import tpu as pltpu

_EPS_NORM = 1e-12
_EPS_D2 = 1e-12
_EPS_LOG = 1e-08
_LOG2E = 1.4426950408889634


def _proj_body(a_ref, b_ref, wa_ref, ba_ref, wb_ref, bb_ref, za_ref, zb_ref):
    xa = jax.lax.dot_general(a_ref[...], wa_ref[...], (((1,), (1,)), ((), ())),
                             preferred_element_type=jnp.float32) + ba_ref[...]
    na = jnp.sqrt(jnp.sum(xa * xa, axis=-1, keepdims=True))
    za_ref[...] = xa / jnp.maximum(na, _EPS_NORM)
    xb = jax.lax.dot_general(b_ref[...], wb_ref[...], (((1,), (1,)), ((), ())),
                             preferred_element_type=jnp.float32) + bb_ref[...]
    nb = jnp.sqrt(jnp.sum(xb * xb, axis=-1, keepdims=True))
    zb_ref[...] = xb / jnp.maximum(nb, _EPS_NORM)


def _main_body(za_ref, zb_ref, zbf_ref, c_ref, out_ref,
               y2_acc, avg_a_acc, avg_b_acc, colsum_acc,
               d2_acc, match_acc, lse_acc, diag_acc):
    i = pl.program_id(0)
    nsteps = pl.num_programs(0)
    bsz = za_ref.shape[0] * nsteps

    @pl.when(i == 0)
    def _init():
        c = c_ref[...]
        ones_m = jnp.ones((8, c.shape[1]), jnp.float32)
        y2_acc[...] = jax.lax.dot_general(
            ones_m, c * c, (((1,), (1,)), ((), ())),
            preferred_element_type=jnp.float32)[:1] + 1.0
        avg_a_acc[...] = jnp.zeros_like(avg_a_acc)
        avg_b_acc[...] = jnp.zeros_like(avg_b_acc)
        colsum_acc[...] = jnp.zeros_like(colsum_acc)
        d2_acc[...] = jnp.zeros_like(d2_acc)
        match_acc[...] = jnp.zeros_like(match_acc)
        lse_acc[...] = jnp.zeros_like(lse_acc)
        diag_acc[...] = jnp.zeros_like(diag_acc)

    y2 = y2_acc[...]
    za = za_ref[...]
    zb = zb_ref[...]
    ones_m8 = jnp.ones((8, za.shape[0]), jnp.float32)

    tr = za.shape[0]
    zm2 = jnp.concatenate([za, zb], axis=0) * (-2.0)
    g2 = jax.lax.dot_general(zm2, c_ref[...], (((1,), (1,)), ((), ())),
                             preferred_element_type=jnp.float32)
    d2c = jnp.maximum(y2 + g2, _EPS_D2)
    m = jnp.min(d2c, axis=-1, keepdims=True)
    e = jnp.exp2(d2c * jax.lax.rsqrt(d2c) * (-5.0 * _LOG2E))
    s = jnp.sum(e, axis=-1, keepdims=True)
    rt = jnp.transpose(1.0 / s, (1, 0))
    avg_a_acc[...] += jax.lax.dot_general(
        rt[:, :tr], e[:tr], (((1,), (0,)), ((), ())),
        preferred_element_type=jnp.float32)
    avg_b_acc[...] += jax.lax.dot_general(
        rt[:, tr:], e[tr:], (((1,), (0,)), ((), ())),
        preferred_element_type=jnp.float32)

    d2_acc[...] += jnp.sum(m, keepdims=True)
    ov = jnp.min(jnp.maximum(d2c[:tr] - m[:tr], d2c[tr:] - m[tr:]), axis=-1,
                 keepdims=True)
    match_acc[...] += jnp.sum((ov <= 0.0).astype(jnp.float32), keepdims=True)

    es = jnp.exp2(jax.lax.dot_general(
        za * jnp.float32(_LOG2E / 0.07), zbf_ref[...],
        (((1,), (1,)), ((), ())), preferred_element_type=jnp.float32))
    rowsum = jnp.sum(es, axis=-1, keepdims=True)
    lse_acc[...] += jnp.sum(jnp.log(rowsum), keepdims=True)
    colsum_acc[...] += jax.lax.dot_general(
        ones_m8, es, (((1,), (0,)), ((), ())),
        preferred_element_type=jnp.float32)[:1]
    diag_acc[...] += jnp.sum(za * zb, keepdims=True) / 0.07

    @pl.when(i == nsteps - 1)
    def _fin():
        avg_a = avg_a_acc[...] * (1.0 / bsz)
        avg_b = avg_b_acc[...] * (1.0 / bsz)
        ent_a = -jnp.sum(avg_a * jnp.log(avg_a + _EPS_LOG), keepdims=True)
        ent_b = -jnp.sum(avg_b * jnp.log(avg_b + _EPS_LOG), keepdims=True)
        col_lse = jnp.sum(jnp.log(colsum_acc[...]), keepdims=True)
        lane = jax.lax.broadcasted_iota(jnp.int32, out_ref.shape, 1)
        out_ref[...] = (jnp.where(lane == 0, d2_acc[...], 0.0)
                        + jnp.where(lane == 1, match_acc[...], 0.0)
                        + jnp.where(lane == 2, ent_a, 0.0)
                        + jnp.where(lane == 3, ent_b, 0.0)
                        + jnp.where(lane == 4, lse_acc[...], 0.0)
                        + jnp.where(lane == 5, col_lse, 0.0)
                        + jnp.where(lane == 6, diag_acc[...], 0.0))


def kernel(a, b, Wa, ba, Wb, bb, codebook):
    bsz, dim_a = a.shape
    dim_b = b.shape[1]
    emb = Wa.shape[0]
    nc = codebook.shape[0]
    f32 = jnp.float32

    tr1 = 1024
    za, zb = pl.pallas_call(
        _proj_body,
        grid=(bsz // tr1,),
        in_specs=[
            pl.BlockSpec((tr1, dim_a), lambda i: (i, 0)),
            pl.BlockSpec((tr1, dim_b), lambda i: (i, 0)),
            pl.BlockSpec((emb, dim_a), lambda i: (0, 0)),
            pl.BlockSpec((1, emb), lambda i: (0, 0)),
            pl.BlockSpec((emb, dim_b), lambda i: (0, 0)),
            pl.BlockSpec((1, emb), lambda i: (0, 0)),
        ],
        out_specs=[
            pl.BlockSpec((tr1, emb), lambda i: (i, 0)),
            pl.BlockSpec((tr1, emb), lambda i: (i, 0)),
        ],
        out_shape=[
            jax.ShapeDtypeStruct((bsz, emb), f32),
            jax.ShapeDtypeStruct((bsz, emb), f32),
        ],
    )(a, b, Wa, ba.reshape(1, emb), Wb, bb.reshape(1, emb))

    tr2 = 256
    main_out = pl.pallas_call(
        _main_body,
        grid=(bsz // tr2,),
        in_specs=[
            pl.BlockSpec((tr2, emb), lambda i: (i, 0)),
            pl.BlockSpec((tr2, emb), lambda i: (i, 0)),
            pl.BlockSpec((bsz, emb), lambda i: (0, 0)),
            pl.BlockSpec((nc, emb), lambda i: (0, 0)),
        ],
        out_specs=pl.BlockSpec((1, 128), lambda i: (0, 0)),
        out_shape=jax.ShapeDtypeStruct((1, 128), f32),
        scratch_shapes=[
            pltpu.VMEM((1, nc), f32),
            pltpu.VMEM((1, nc), f32),
            pltpu.VMEM((1, nc), f32),
            pltpu.VMEM((1, bsz), f32),
            pltpu.VMEM((1, 1), f32),
            pltpu.VMEM((1, 1), f32),
            pltpu.VMEM((1, 1), f32),
            pltpu.VMEM((1, 1), f32),
        ],
    )(za, zb, zb, codebook)

    d2min_sum = main_out[0, 0]
    match_sum = main_out[0, 1]
    ent_a = main_out[0, 2]
    ent_b = main_out[0, 3]
    lse_row_sum = main_out[0, 4]
    col_lse_sum = main_out[0, 5]
    diag_sum = main_out[0, 6]

    rec = 1.25 * d2min_sum / (bsz * emb)
    cm = 0.5 * (lse_row_sum + col_lse_sum) / bsz - diag_sum / bsz
    div = 0.5 * (ent_a + ent_b)
    loss = rec + 0.5 * cm - 0.1 * div
    match = match_sum / bsz
    return (loss, match)
